# baseline (speedup 1.0000x reference)
import functools

import jax
import jax.numpy as jnp
from jax import lax
from jax.experimental import pallas as pl
from jax.experimental.pallas import tpu as pltpu
from jax.experimental.pallas import tpu_sc as plsc

B, P, C = 4096, 50, 256
NC, NS = 2, 16
NW = NC * NS
BPW = B // NW
NBC = 64
KPB = BPW // NBC
NCHK = P * KPB
NBUF = 4

_mesh = plsc.VectorSubcoreMesh(core_axis_name="c", subcore_axis_name="s")


@functools.partial(
    pl.kernel,
    out_type=jax.ShapeDtypeStruct((P, B, C), jnp.float32),
    mesh=_mesh,
    compiler_params=pltpu.CompilerParams(needs_layout_passes=False),
    scratch_types=[pltpu.VMEM((P, BPW), jnp.int32)]
    + [pltpu.VMEM((NBC, C), jnp.float32)] * NBUF
    + [pltpu.SemaphoreType.DMA] * NBUF,
)
def _one_hot_sc(xt_hbm, out_hbm, idx_v, *bufs_sems):
    bufs = bufs_sems[:NBUF]
    sems = bufs_sems[NBUF:]
    cid = lax.axis_index("c")
    sid = lax.axis_index("s")
    wid = sid * NC + cid
    b0 = wid * BPW

    pltpu.sync_copy(xt_hbm.at[:, pl.ds(b0, BPW)], idx_v)

    zeros16 = jnp.zeros((16,), jnp.float32)
    ones16 = jnp.ones((16,), jnp.float32)
    iota16 = lax.iota(jnp.int32, 16)

    def _zero(i, carry):
        for b in range(NBUF):
            for k in range(C // 16):
                bufs[b][i, pl.ds(k * 16, 16)] = zeros16
        return carry

    lax.fori_loop(0, NBC, _zero, 0)

    def _scatter(buf, c, val):
        p = c // KPB
        k = c % KPB
        for j in range(NBC // 16):
            row16 = j * 16 + iota16
            idx16 = idx_v[p, pl.ds(k * NBC + j * 16, 16)]
            plsc.store_scatter(buf, [row16, idx16], val)

    def _dst(c):
        p = c // KPB
        k = c % KPB
        return out_hbm.at[p, pl.ds(b0 + k * NBC, NBC), :]

    for b in range(NBUF):
        _scatter(bufs[b], b, ones16)
        pltpu.async_copy(bufs[b], _dst(b), sems[b])

    def _grp(g, carry):
        for b in range(NBUF):
            c = g * NBUF + b
            pltpu.make_async_copy(bufs[b], _dst(c - NBUF), sems[b]).wait()
            _scatter(bufs[b], c - NBUF, zeros16)
            _scatter(bufs[b], c, ones16)
            pltpu.async_copy(bufs[b], _dst(c), sems[b])
        return carry

    lax.fori_loop(1, NCHK // NBUF, _grp, 0)

    for b in range(NBUF):
        pltpu.make_async_copy(bufs[b], _dst(NCHK - NBUF + b), sems[b]).wait()


def kernel(x):
    out_t = _one_hot_sc(x.T)
    return out_t.transpose(1, 0, 2)


# speedup vs baseline: 1.0154x; 1.0154x over previous
import functools

import jax
import jax.numpy as jnp
from jax import lax
from jax.experimental import pallas as pl
from jax.experimental.pallas import tpu as pltpu
from jax.experimental.pallas import tpu_sc as plsc

B, P, C = 4096, 50, 256
NC, NS = 2, 16
NW = NC * NS
BPW = B // NW
NBC = 64
KPB = BPW // NBC
NCHK = P * KPB
NBUF = 2

_mesh = plsc.VectorSubcoreMesh(core_axis_name="c", subcore_axis_name="s")


@functools.partial(
    pl.kernel,
    out_type=jax.ShapeDtypeStruct((P, B, C), jnp.float32),
    mesh=_mesh,
    compiler_params=pltpu.CompilerParams(needs_layout_passes=False),
    scratch_types=[
        pltpu.VMEM((P, BPW), jnp.int32),
        pltpu.VMEM((NBC, C), jnp.float32),
        pltpu.VMEM((NBC, C), jnp.float32),
        pltpu.SemaphoreType.DMA,
        pltpu.SemaphoreType.DMA,
    ],
)
def _one_hot_sc(xt_hbm, out_hbm, idx_v, buf0, buf1, sem0, sem1):
    cid = lax.axis_index("c")
    sid = lax.axis_index("s")
    wid = sid * NC + cid
    b0 = wid * BPW

    pltpu.sync_copy(xt_hbm.at[:, pl.ds(b0, BPW)], idx_v)

    zeros16 = jnp.zeros((16,), jnp.float32)
    ones16 = jnp.ones((16,), jnp.float32)
    iota16 = lax.iota(jnp.int32, 16)
    bufs = (buf0, buf1)
    sems = (sem0, sem1)

    def _zero(i, carry):
        for b in range(NBUF):
            for k in range(C // 16):
                bufs[b][i, pl.ds(k * 16, 16)] = zeros16
        return carry

    lax.fori_loop(0, NBC, _zero, 0)

    def _scatter(buf, c, val):
        p = c // KPB
        k = c % KPB
        for j in range(NBC // 16):
            row16 = j * 16 + iota16
            idx16 = idx_v[p, pl.ds(k * NBC + j * 16, 16)]
            plsc.store_scatter(buf, [row16, idx16], val)

    def _dst(c):
        p = c // KPB
        k = c % KPB
        return out_hbm.at[p, pl.ds(b0 + k * NBC, NBC), :]

    for b in range(NBUF):
        _scatter(bufs[b], b, ones16)
        pltpu.async_copy(bufs[b], _dst(b), sems[b])

    def _grp(g, carry):
        for b in range(NBUF):
            c = g * NBUF + b
            pltpu.make_async_copy(bufs[b], _dst(c - NBUF), sems[b]).wait()
            _scatter(bufs[b], c - NBUF, zeros16)
            _scatter(bufs[b], c, ones16)
            pltpu.async_copy(bufs[b], _dst(c), sems[b])
        return carry

    lax.fori_loop(1, NCHK // NBUF, _grp, 0)

    for b in range(NBUF):
        pltpu.make_async_copy(bufs[b], _dst(NCHK - NBUF + b), sems[b]).wait()


def kernel(x):
    out_t = _one_hot_sc(x.T)
    return out_t.transpose(1, 0, 2)


# speedup vs baseline: 1.0409x; 1.0252x over previous
import functools

import jax
import jax.numpy as jnp
from jax import lax
from jax.experimental import pallas as pl
from jax.experimental.pallas import tpu as pltpu
from jax.experimental.pallas import tpu_sc as plsc

B, P, C = 4096, 50, 256
NC, NS = 2, 16
NW = NC * NS
BPW = B // NW
NBC = 32
KPB = BPW // NBC
NCHK = P * KPB
NBUF = 2

_mesh = plsc.VectorSubcoreMesh(core_axis_name="c", subcore_axis_name="s")


@functools.partial(
    pl.kernel,
    out_type=jax.ShapeDtypeStruct((P, B, C), jnp.float32),
    mesh=_mesh,
    compiler_params=pltpu.CompilerParams(needs_layout_passes=False),
    scratch_types=[
        pltpu.VMEM((P, BPW), jnp.int32),
        pltpu.VMEM((NBC, C), jnp.float32),
        pltpu.VMEM((NBC, C), jnp.float32),
        pltpu.SemaphoreType.DMA,
        pltpu.SemaphoreType.DMA,
    ],
)
def _one_hot_sc(xt_hbm, out_hbm, idx_v, buf0, buf1, sem0, sem1):
    cid = lax.axis_index("c")
    sid = lax.axis_index("s")
    wid = sid * NC + cid
    b0 = wid * BPW

    pltpu.sync_copy(xt_hbm.at[:, pl.ds(b0, BPW)], idx_v)

    zeros16 = jnp.zeros((16,), jnp.float32)
    ones16 = jnp.ones((16,), jnp.float32)
    iota16 = lax.iota(jnp.int32, 16)
    bufs = (buf0, buf1)
    sems = (sem0, sem1)

    def _zero(i, carry):
        for b in range(NBUF):
            for k in range(C // 16):
                bufs[b][i, pl.ds(k * 16, 16)] = zeros16
        return carry

    lax.fori_loop(0, NBC, _zero, 0)

    def _scatter(buf, c, val):
        p = c // KPB
        k = c % KPB
        for j in range(NBC // 16):
            row16 = j * 16 + iota16
            idx16 = idx_v[p, pl.ds(k * NBC + j * 16, 16)]
            plsc.store_scatter(buf, [row16, idx16], val)

    def _dst(c):
        p = c // KPB
        k = c % KPB
        return out_hbm.at[p, pl.ds(b0 + k * NBC, NBC), :]

    for b in range(NBUF):
        _scatter(bufs[b], b, ones16)
        pltpu.async_copy(bufs[b], _dst(b), sems[b])

    def _grp(g, carry):
        for b in range(NBUF):
            c = g * NBUF + b
            pltpu.make_async_copy(bufs[b], _dst(c - NBUF), sems[b]).wait()
            _scatter(bufs[b], c - NBUF, zeros16)
            _scatter(bufs[b], c, ones16)
            pltpu.async_copy(bufs[b], _dst(c), sems[b])
        return carry

    lax.fori_loop(1, NCHK // NBUF, _grp, 0)

    for b in range(NBUF):
        pltpu.make_async_copy(bufs[b], _dst(NCHK - NBUF + b), sems[b]).wait()


def kernel(x):
    out_t = _one_hot_sc(x.T)
    return out_t.transpose(1, 0, 2)
